# baseline (speedup 1.0000x reference)
import functools

import jax
import jax.numpy as jnp
from jax.experimental import pallas as pl
from jax.experimental.pallas import tpu as pltpu
from jax.experimental.pallas import tpu_sc as plsc

N_STATES = 100000
N_SAMPLES = 1000
N_WARMUP = 100
STEPS_PER_SAMPLE = 5

LANES = 128
CHUNK_SUB = 56
N_CHUNKS = 14
ROWS_PER_STEP = 10
N_STEPS = N_SAMPLES // ROWS_PER_STEP
SUBROWS = CHUNK_SUB * N_CHUNKS
PADDED = SUBROWS * LANES

KS0 = 64467757
KS1 = 2916123636
KS2 = (KS0 ^ KS1 ^ 0x1BD11BDA) & 0xFFFFFFFF

_ROT_A = (13, 15, 26, 6)
_ROT_B = (17, 29, 16, 24)


SC_LANES = 16
PAD_SAMPLES = 1008
HIST_PAD = N_STATES + SC_LANES


VREGS_PER_CHUNK = CHUNK_SUB // 8


def _gumbel_scores(flat_u, base_ks1, lwc):
    ks = (KS0, KS1, KS2)
    x0 = jnp.full(flat_u.shape, jnp.uint32(KS0), dtype=jnp.uint32)
    x1 = flat_u + base_ks1
    rots = (_ROT_A, _ROT_B)
    for rnd in range(5):
        for r in rots[rnd % 2]:
            x0 = x0 + x1
            x1 = jax.lax.shift_left(x1, jnp.uint32(r)) | \
                jax.lax.shift_right_logical(x1, jnp.uint32(32 - r))
            x1 = x0 ^ x1
        x0 = x0 + jnp.uint32(ks[(rnd + 1) % 3])
        x1 = x1 + jnp.uint32((ks[(rnd + 2) % 3] + rnd + 1) & 0xFFFFFFFF)
    bits = x0 ^ x1

    fb = jax.lax.shift_right_logical(bits, jnp.uint32(9)) | \
        jnp.uint32(0x3F800000)
    u = jax.lax.bitcast_convert_type(fb, jnp.float32) - jnp.float32(1.0)
    tiny = jnp.float32(jnp.finfo(jnp.float32).tiny)
    u = u + tiny
    g = -jnp.log(-jnp.log(u))
    return g + lwc


def _sampler_kernel(lw_ref, pv_ref, pj_ref):
    p = pl.program_id(0)

    @pl.when(p == 0)
    def _init():
        pv_ref[...] = jnp.full_like(pv_ref, -jnp.inf)
        pj_ref[...] = jnp.full_like(pj_ref, N_STATES)

    base_ks1 = [
        ((N_WARMUP + STEPS_PER_SAMPLE * (ROWS_PER_STEP * p + r)) * N_STATES)
        .astype(jnp.uint32) + jnp.uint32(KS1)
        for r in range(ROWS_PER_STEP)
    ]

    i = jax.lax.broadcasted_iota(jnp.int32, (CHUNK_SUB, LANES), 0)
    c = jax.lax.broadcasted_iota(jnp.int32, (CHUNK_SUB, LANES), 1)
    flat0 = i * LANES + c

    def chunk(k, carry):
        lwc = lw_ref[pl.ds(k * CHUNK_SUB, CHUNK_SUB), :]
        flat = flat0 + k * (CHUNK_SUB * LANES)
        flat_u = flat.astype(jnp.uint32)
        f3 = flat.reshape(VREGS_PER_CHUNK, 8, LANES)
        out = []
        for r in range(ROWS_PER_STEP):
            best_v, best_j = carry[2 * r], carry[2 * r + 1]
            score = _gumbel_scores(flat_u, base_ks1[r], lwc)
            s3 = score.reshape(VREGS_PER_CHUNK, 8, LANES)
            for v in range(VREGS_PER_CHUNK):
                upd = s3[v] > best_v
                best_v = jnp.where(upd, s3[v], best_v)
                best_j = jnp.where(upd, f3[v], best_j)
            out += [best_v, best_j]
        return tuple(out)

    neg_inf = jnp.full((8, LANES), -jnp.inf, dtype=jnp.float32)
    zero_j = jnp.zeros((8, LANES), dtype=jnp.int32)
    carry = (neg_inf, zero_j) * ROWS_PER_STEP
    for k in range(N_CHUNKS):
        carry = chunk(k, carry)

    pvs, pjs = [], []
    for r in range(ROWS_PER_STEP):
        best_v, best_j = carry[2 * r], carry[2 * r + 1]
        bv_max = jnp.max(best_v, axis=0, keepdims=True)
        eq = best_v == bv_max
        pvs.append(bv_max)
        pjs.append(jnp.min(jnp.where(eq, best_j, jnp.int32(2**30)), axis=0,
                           keepdims=True))
    row0 = ROWS_PER_STEP * p
    pv_ref[pl.ds(row0, ROWS_PER_STEP), :] = jnp.concatenate(pvs, axis=0)
    pj_ref[pl.ds(row0, ROWS_PER_STEP), :] = jnp.concatenate(pjs, axis=0)


def _draw_partials(lw_pad):
    return pl.pallas_call(
        _sampler_kernel,
        grid=(N_STEPS,),
        in_specs=[pl.BlockSpec((SUBROWS, LANES), lambda p: (0, 0))],
        out_specs=[
            pl.BlockSpec((PAD_SAMPLES, LANES), lambda p: (0, 0)),
            pl.BlockSpec((PAD_SAMPLES, LANES), lambda p: (0, 0)),
        ],
        out_shape=[
            jax.ShapeDtypeStruct((PAD_SAMPLES, LANES), jnp.float32),
            jax.ShapeDtypeStruct((PAD_SAMPLES, LANES), jnp.int32),
        ],
    )(lw_pad)


def _lane_argmax_kernel(pv_ref, pj_ref, idx_ref):
    v = pv_ref[...]
    j = pj_ref[...]
    m = jnp.max(v, axis=1, keepdims=True)
    wj = jnp.min(jnp.where(v == m, j, jnp.int32(2**30)), axis=1, keepdims=True)
    idx_ref[...] = wj


def _lane_argmax(pv, pj):
    return pl.pallas_call(
        _lane_argmax_kernel,
        in_specs=[
            pl.BlockSpec((PAD_SAMPLES, LANES), lambda: (0, 0)),
            pl.BlockSpec((PAD_SAMPLES, LANES), lambda: (0, 0)),
        ],
        out_specs=pl.BlockSpec((PAD_SAMPLES, 1), lambda: (0, 0)),
        out_shape=jax.ShapeDtypeStruct((PAD_SAMPLES, 1), jnp.int32),
    )(pv, pj)


@functools.cache
def _sc_histogram_fn():
    @functools.partial(
        pl.kernel,
        out_type=jax.ShapeDtypeStruct((N_STATES,), jnp.float32),
        mesh=plsc.VectorSubcoreMesh(core_axis_name="c", subcore_axis_name="s"),
        compiler_params=pltpu.CompilerParams(needs_layout_passes=False),
        scratch_types=[
            pltpu.VMEM((HIST_PAD,), jnp.float32),
            pltpu.VMEM((PAD_SAMPLES,), jnp.int32),
        ],
    )
    def _sc_histogram(idx_hbm, zeros_hbm, out_hbm, counts_v, idx_v):
        wid = jax.lax.axis_index("s") * 2 + jax.lax.axis_index("c")

        @pl.when(wid == 0)
        def _():
            pltpu.sync_copy(zeros_hbm, counts_v)
            pltpu.sync_copy(idx_hbm, idx_v)
            ones = jnp.full((SC_LANES,), 1.0, dtype=jnp.float32)
            for i in range(PAD_SAMPLES // SC_LANES):
                v = idx_v[pl.ds(i * SC_LANES, SC_LANES)]
                plsc.addupdate_scatter(counts_v, [v], ones)
            pltpu.sync_copy(counts_v.at[pl.ds(0, N_STATES)], out_hbm)

    return _sc_histogram


def kernel(A, D, observation):
    likelihood = A[observation, :]
    posterior_weights = likelihood * D
    posterior_weights = posterior_weights / (jnp.sum(posterior_weights) + 1e-16)
    log_weights = jnp.log(posterior_weights + 1e-16)
    lw_pad = jnp.concatenate(
        [log_weights,
         jnp.full((PADDED - N_STATES,), -jnp.inf, dtype=jnp.float32)]
    ).reshape(SUBROWS, LANES)

    pv, pj = _draw_partials(lw_pad)
    idx = _lane_argmax(pv, pj).reshape(PAD_SAMPLES)
    counts = _sc_histogram_fn()(idx, jnp.zeros((HIST_PAD,), jnp.float32))

    posterior_estimate = counts / float(N_SAMPLES)
    return posterior_estimate / (jnp.sum(posterior_estimate) + 1e-16)


# speedup vs baseline: 1.0038x; 1.0038x over previous
import functools

import jax
import jax.numpy as jnp
from jax.experimental import pallas as pl
from jax.experimental.pallas import tpu as pltpu
from jax.experimental.pallas import tpu_sc as plsc

N_STATES = 100000
N_SAMPLES = 1000
N_WARMUP = 100
STEPS_PER_SAMPLE = 5

LANES = 128
CHUNK_SUB = 56
N_CHUNKS = 14
ROWS_PER_STEP = 5
N_STEPS = N_SAMPLES // ROWS_PER_STEP
SUBROWS = CHUNK_SUB * N_CHUNKS
PADDED = SUBROWS * LANES

KS0 = 64467757
KS1 = 2916123636
KS2 = (KS0 ^ KS1 ^ 0x1BD11BDA) & 0xFFFFFFFF

_ROT_A = (13, 15, 26, 6)
_ROT_B = (17, 29, 16, 24)


SC_LANES = 16
PAD_SAMPLES = 1008
HIST_PAD = N_STATES + SC_LANES


VREGS_PER_CHUNK = CHUNK_SUB // 8


def _gumbel_scores(flat_u, base_ks1, lwc):
    ks = (KS0, KS1, KS2)
    x0 = jnp.full(flat_u.shape, jnp.uint32(KS0), dtype=jnp.uint32)
    x1 = flat_u + base_ks1
    rots = (_ROT_A, _ROT_B)
    for rnd in range(5):
        for r in rots[rnd % 2]:
            x0 = x0 + x1
            x1 = jax.lax.shift_left(x1, jnp.uint32(r)) | \
                jax.lax.shift_right_logical(x1, jnp.uint32(32 - r))
            x1 = x0 ^ x1
        x0 = x0 + jnp.uint32(ks[(rnd + 1) % 3])
        x1 = x1 + jnp.uint32((ks[(rnd + 2) % 3] + rnd + 1) & 0xFFFFFFFF)
    bits = x0 ^ x1

    fb = jax.lax.shift_right_logical(bits, jnp.uint32(9)) | \
        jnp.uint32(0x3F800000)
    u = jax.lax.bitcast_convert_type(fb, jnp.float32) - jnp.float32(1.0)
    tiny = jnp.float32(jnp.finfo(jnp.float32).tiny)
    u = u + tiny
    g = -jnp.log(-jnp.log(u))
    return g + lwc


def _sampler_kernel(lw_ref, pv_ref, pj_ref):
    p = pl.program_id(0)

    @pl.when(p == 0)
    def _init():
        pv_ref[...] = jnp.full_like(pv_ref, -jnp.inf)
        pj_ref[...] = jnp.full_like(pj_ref, N_STATES)

    base_ks1 = [
        ((N_WARMUP + STEPS_PER_SAMPLE * (ROWS_PER_STEP * p + r)) * N_STATES)
        .astype(jnp.uint32) + jnp.uint32(KS1)
        for r in range(ROWS_PER_STEP)
    ]

    i = jax.lax.broadcasted_iota(jnp.int32, (CHUNK_SUB, LANES), 0)
    c = jax.lax.broadcasted_iota(jnp.int32, (CHUNK_SUB, LANES), 1)
    flat0 = i * LANES + c

    def chunk(k, carry):
        lwc = lw_ref[pl.ds(k * CHUNK_SUB, CHUNK_SUB), :]
        flat = flat0 + k * (CHUNK_SUB * LANES)
        flat_u = flat.astype(jnp.uint32)
        f3 = flat.reshape(VREGS_PER_CHUNK, 8, LANES)
        out = []
        for r in range(ROWS_PER_STEP):
            best_v, best_j = carry[2 * r], carry[2 * r + 1]
            score = _gumbel_scores(flat_u, base_ks1[r], lwc)
            s3 = score.reshape(VREGS_PER_CHUNK, 8, LANES)
            for v in range(VREGS_PER_CHUNK):
                upd = s3[v] > best_v
                best_v = jnp.where(upd, s3[v], best_v)
                best_j = jnp.where(upd, f3[v], best_j)
            out += [best_v, best_j]
        return tuple(out)

    neg_inf = jnp.full((8, LANES), -jnp.inf, dtype=jnp.float32)
    zero_j = jnp.zeros((8, LANES), dtype=jnp.int32)
    carry = (neg_inf, zero_j) * ROWS_PER_STEP
    for k in range(N_CHUNKS):
        carry = chunk(k, carry)

    pvs, pjs = [], []
    for r in range(ROWS_PER_STEP):
        best_v, best_j = carry[2 * r], carry[2 * r + 1]
        bv_max = jnp.max(best_v, axis=0, keepdims=True)
        eq = best_v == bv_max
        pvs.append(bv_max)
        pjs.append(jnp.min(jnp.where(eq, best_j, jnp.int32(2**30)), axis=0,
                           keepdims=True))
    row0 = ROWS_PER_STEP * p
    pv_ref[pl.ds(row0, ROWS_PER_STEP), :] = jnp.concatenate(pvs, axis=0)
    pj_ref[pl.ds(row0, ROWS_PER_STEP), :] = jnp.concatenate(pjs, axis=0)


def _draw_partials(lw_pad):
    return pl.pallas_call(
        _sampler_kernel,
        grid=(N_STEPS,),
        in_specs=[pl.BlockSpec((SUBROWS, LANES), lambda p: (0, 0))],
        out_specs=[
            pl.BlockSpec((PAD_SAMPLES, LANES), lambda p: (0, 0)),
            pl.BlockSpec((PAD_SAMPLES, LANES), lambda p: (0, 0)),
        ],
        out_shape=[
            jax.ShapeDtypeStruct((PAD_SAMPLES, LANES), jnp.float32),
            jax.ShapeDtypeStruct((PAD_SAMPLES, LANES), jnp.int32),
        ],
    )(lw_pad)


def _lane_argmax_kernel(pv_ref, pj_ref, idx_ref):
    v = pv_ref[...]
    j = pj_ref[...]
    m = jnp.max(v, axis=1, keepdims=True)
    wj = jnp.min(jnp.where(v == m, j, jnp.int32(2**30)), axis=1, keepdims=True)
    idx_ref[...] = wj


def _lane_argmax(pv, pj):
    return pl.pallas_call(
        _lane_argmax_kernel,
        in_specs=[
            pl.BlockSpec((PAD_SAMPLES, LANES), lambda: (0, 0)),
            pl.BlockSpec((PAD_SAMPLES, LANES), lambda: (0, 0)),
        ],
        out_specs=pl.BlockSpec((PAD_SAMPLES, 1), lambda: (0, 0)),
        out_shape=jax.ShapeDtypeStruct((PAD_SAMPLES, 1), jnp.int32),
    )(pv, pj)


@functools.cache
def _sc_histogram_fn():
    @functools.partial(
        pl.kernel,
        out_type=jax.ShapeDtypeStruct((N_STATES,), jnp.float32),
        mesh=plsc.VectorSubcoreMesh(core_axis_name="c", subcore_axis_name="s"),
        compiler_params=pltpu.CompilerParams(needs_layout_passes=False),
        scratch_types=[
            pltpu.VMEM((HIST_PAD,), jnp.float32),
            pltpu.VMEM((PAD_SAMPLES,), jnp.int32),
        ],
    )
    def _sc_histogram(idx_hbm, zeros_hbm, out_hbm, counts_v, idx_v):
        wid = jax.lax.axis_index("s") * 2 + jax.lax.axis_index("c")

        @pl.when(wid == 0)
        def _():
            pltpu.sync_copy(zeros_hbm, counts_v)
            pltpu.sync_copy(idx_hbm, idx_v)
            ones = jnp.full((SC_LANES,), 1.0, dtype=jnp.float32)
            for i in range(PAD_SAMPLES // SC_LANES):
                v = idx_v[pl.ds(i * SC_LANES, SC_LANES)]
                plsc.addupdate_scatter(counts_v, [v], ones)
            pltpu.sync_copy(counts_v.at[pl.ds(0, N_STATES)], out_hbm)

    return _sc_histogram


def kernel(A, D, observation):
    likelihood = A[observation, :]
    posterior_weights = likelihood * D
    posterior_weights = posterior_weights / (jnp.sum(posterior_weights) + 1e-16)
    log_weights = jnp.log(posterior_weights + 1e-16)
    lw_pad = jnp.concatenate(
        [log_weights,
         jnp.full((PADDED - N_STATES,), -jnp.inf, dtype=jnp.float32)]
    ).reshape(SUBROWS, LANES)

    pv, pj = _draw_partials(lw_pad)
    idx = _lane_argmax(pv, pj).reshape(PAD_SAMPLES)
    counts = _sc_histogram_fn()(idx, jnp.zeros((HIST_PAD,), jnp.float32))

    posterior_estimate = counts / float(N_SAMPLES)
    return posterior_estimate / (jnp.sum(posterior_estimate) + 1e-16)
